# baseline (speedup 1.0000x reference)
import math

import jax
import jax.numpy as jnp
from jax import lax
from jax.experimental import pallas as pl
from jax.experimental.pallas import tpu as pltpu
from jax.experimental.pallas import tpu_sc as plsc

_LS = 0.1
_V = 100000
_B = 1024
_CONF = 1.0 - _LS
_SMOOTH = _LS / (_V - 2)
_C0 = _B * ((_V - 1) * _SMOOTH * math.log(_SMOOTH) + _CONF * math.log(_CONF))

_NW = 32
_RPW = 16
_RSC = _NW * _RPW
_RTC = _B - _RSC

_BN = 4096
_NBLK = (_V + _BN - 1) // _BN

_LANES = 16
_WCH = 9088
_NCH = 11
_CSP = _NCH * _WCH
_SLIV = _V - _CSP
_ACCS = 8
_CHUNK = _WCH // (_LANES * _ACCS)


def _tc_kernel(t_ref, ts_ref, x_ref, sliv_ref, out_ref, acc_ref):
    j = pl.program_id(0)

    @pl.when(j == 0)
    def _init():
        sl = sliv_ref[...]
        scol = _CSP + lax.broadcasted_iota(jnp.int32, sl.shape, 1)
        acc_ref[0] = jnp.sum(jnp.where(scol < _V, sl, 0.0))
        acc_ref[1] = jnp.sum(jnp.where(scol == ts_ref[...], sl, 0.0))

    x = x_ref[...]
    col = j * _BN + lax.broadcasted_iota(jnp.int32, x.shape, 1)
    acc_ref[0] += jnp.sum(jnp.where(col < _V, x, 0.0))
    acc_ref[1] += jnp.sum(jnp.where(col == t_ref[...], x, 0.0))

    @pl.when(j == _NBLK - 1)
    def _fin():
        out_ref[0] = (_SMOOTH * acc_ref[0]
                      + (_CONF - _SMOOTH) * acc_ref[1]).astype(jnp.float32)


def _tc_partial(output, t2d):
    return pl.pallas_call(
        _tc_kernel,
        grid=(_NBLK,),
        in_specs=[
            pl.BlockSpec((_RTC, 1), lambda j: (0, 0)),
            pl.BlockSpec((_RSC, 1), lambda j: (_RTC // _RSC, 0)),
            pl.BlockSpec((_RTC, _BN), lambda j: (0, j)),
            pl.BlockSpec((_RSC, 128), lambda j: (_RTC // _RSC, _CSP // 128)),
        ],
        out_specs=pl.BlockSpec(memory_space=pltpu.SMEM),
        out_shape=jax.ShapeDtypeStruct((1,), jnp.float32),
        scratch_shapes=[pltpu.SMEM((2,), jnp.float32)],
    )(t2d, t2d, output, output)


def _sc_body(x_hbm, t_hbm, out_hbm, buf_v, tgt_v, res_v, sem):
    wid = lax.axis_index("s") * 2 + lax.axis_index("c")
    rbase = _RTC + wid * _RPW

    pltpu.sync_copy(t_hbm.at[pl.ds(rbase, _RPW)], tgt_v)
    t16 = tgt_v[...]

    iota = lax.iota(jnp.int32, _LANES)
    zero = jnp.zeros((_LANES,), jnp.float32)
    acc_s = zero
    acc_g = zero
    for g in range(_RPW // 8):
        row0 = rbase + g * 8
        for c in range(_NCH):
            c0 = c * _WCH
            pltpu.async_copy(
                x_hbm.at[pl.ds(row0, 8), pl.ds(c0, _WCH)], buf_v, sem).wait()
            for r in range(8):
                row_ref = buf_v.at[r]

                def body(i, accs, row_ref=row_ref):
                    return tuple(
                        accs[u] + row_ref[pl.ds((i * _ACCS + u) * _LANES,
                                                _LANES)]
                        for u in range(_ACCS))

                accs = lax.fori_loop(0, _CHUNK, body, (zero,) * _ACCS)
                row_sum = accs[0]
                for u in range(1, _ACCS):
                    row_sum = row_sum + accs[u]
                acc_s = acc_s + row_sum

                t_r = t16[g * 8 + r]
                rel = t_r - c0
                inb = (rel >= 0) & (rel < _WCH)
                start = pl.multiple_of(
                    jnp.clip((rel >> 4) << 4, 0, _WCH - _LANES), _LANES)
                gv = row_ref[pl.ds(start, _LANES)]
                lane = jnp.where(inb, rel & (_LANES - 1), -1)
                acc_g = acc_g + jnp.where(iota == lane, gv, 0.0)

    res_v[...] = _SMOOTH * acc_s + (_CONF - _SMOOTH) * acc_g
    pltpu.sync_copy(res_v, out_hbm.at[pl.ds(wid * _LANES, _LANES)])


_sc_partial = pl.kernel(
    _sc_body,
    mesh=plsc.VectorSubcoreMesh(core_axis_name="c", subcore_axis_name="s"),
    out_type=jax.ShapeDtypeStruct((_NW * _LANES,), jnp.float32),
    scratch_types=[
        pltpu.VMEM((8, _WCH), jnp.float32),
        pltpu.VMEM((_RPW,), jnp.int32),
        pltpu.VMEM((_LANES,), jnp.float32),
        pltpu.SemaphoreType.DMA,
    ],
)


def kernel(output, target):
    t32 = target.astype(jnp.int32)
    tc_out = _tc_partial(output, t32.reshape(_B, 1))
    sc_out = _sc_partial(output, t32)
    return (_C0 - tc_out[0] - jnp.sum(sc_out)).astype(jnp.float32)
